# baseline (speedup 1.0000x reference)
import functools

import jax
import jax.numpy as jnp
from jax import lax
from jax.experimental import pallas as pl
from jax.experimental.pallas import tpu as pltpu
from jax.experimental.pallas import tpu_sc as plsc

N = 10000
E = 160000
H = 256

NB = 1000
NG = N // NB

NTILES = 16
EPT = E // NTILES
CH = 40
CHP = 48
NCH = EPT // CH
NP = 10240
RPT = NP // NTILES
ZR = 32
UW = 128


def _embed_body(x_ref, at_ref, ct_ref, ht_ref, h_ref, ids_ref):
    xb = x_ref[...]
    a_id = jnp.clip(xb[:, 0:1].astype(jnp.int32), 0, 10)
    c_id = jnp.clip(xb[:, 2:3].astype(jnp.int32) + 3, 0, 6)
    y_id = jnp.clip(xb[:, 1:2].astype(jnp.int32), 0, 7)

    def onehot(ids, k):
        i = lax.broadcasted_iota(jnp.int32, (NB, k), 1)
        return (ids == i).astype(jnp.float32)

    a = onehot(a_id, 11) @ at_ref[...]
    c = onehot(c_id, 7) @ ct_ref[...]
    y = onehot(y_id, 8) @ ht_ref[...]
    h_ref[...] = jnp.concatenate(
        [a, c, y, jnp.zeros((NB, 128), jnp.float32)], axis=1)
    ids_ref[...] = a_id


def _embed(x, atom_table, charge_table, hybrid_table):
    h0, ids = pl.pallas_call(
        _embed_body,
        grid=(NG,),
        in_specs=[
            pl.BlockSpec((NB, 3), lambda i: (i, 0)),
            pl.BlockSpec((11, 64), lambda i: (0, 0)),
            pl.BlockSpec((7, 32), lambda i: (0, 0)),
            pl.BlockSpec((8, 32), lambda i: (0, 0)),
        ],
        out_specs=[
            pl.BlockSpec((NB, 256), lambda i: (i, 0)),
            pl.BlockSpec((NB, 1), lambda i: (i, 0)),
        ],
        out_shape=[
            jax.ShapeDtypeStruct((N, 256), jnp.float32),
            jax.ShapeDtypeStruct((N, 1), jnp.int32),
        ],
    )(x, atom_table, charge_table, hybrid_table)
    return h0, ids


def _tables_body(bt_ref, wn1_ref, bn1_ref, wa_ref, ba_ref, ctab_ref):
    btab = bt_ref[...]
    for i in range(3):
        C = btab @ wn1_ref[i, 2 * H:, :] + bn1_ref[i:i + 1, :]
        Cp = jnp.concatenate([C, jnp.zeros((3, 256), jnp.float32)], 0)
        bw = jax.nn.sigmoid(btab @ wa_ref[i] + ba_ref[i:i + 1, :])
        bwp = jnp.broadcast_to(
            jnp.concatenate([bw, jnp.zeros((3, 1), jnp.float32)], 0), (8, 128))
        for c in range(2):
            ctab_ref[i, pl.ds(8 * c, 8)] = jnp.concatenate(
                [Cp[:, 128 * c:128 * c + 128], bwp], axis=1)


def _tables(bond_table, Wn1, bn1, Wa, ba):
    return pl.pallas_call(
        _tables_body,
        out_shape=jax.ShapeDtypeStruct((3, 16, 256), jnp.float32),
    )(bond_table, Wn1, bn1, Wa, ba)


def _bt_body(ea_ref, bt_ref):
    bt_ref[...] = jnp.clip(ea_ref[...][:, 0:1].astype(jnp.int32), 0, 4)


def _bt(edge_attr):
    return pl.pallas_call(
        _bt_body,
        grid=(E // NB,),
        in_specs=[pl.BlockSpec((NB, 4), lambda i: (i, 0))],
        out_specs=pl.BlockSpec((NB, 1), lambda i: (i, 0)),
        out_shape=jax.ShapeDtypeStruct((E, 1), jnp.int32),
    )(edge_attr)


def _pq_body(h_ref, wa_ref, wb_ref, p_ref, q_ref):
    hb = h_ref[...]
    p_ref[...] = jnp.dot(hb, wa_ref[...], preferred_element_type=jnp.float32)
    q_ref[...] = jnp.dot(hb, wb_ref[...], preferred_element_type=jnp.float32)


def _pq(h, W1a, W1b):
    return pl.pallas_call(
        _pq_body,
        grid=(NG, 2),
        in_specs=[
            pl.BlockSpec((NB, 256), lambda i, j: (i, 0)),
            pl.BlockSpec((256, 128), lambda i, j: (0, j)),
            pl.BlockSpec((256, 128), lambda i, j: (0, j)),
        ],
        out_specs=[
            pl.BlockSpec((NB, 128), lambda i, j: (j * NG + i, 0)),
            pl.BlockSpec((NB, 128), lambda i, j: (j * NG + i, 0)),
        ],
        out_shape=[
            jax.ShapeDtypeStruct((2 * N, 128), jnp.float32),
            jax.ShapeDtypeStruct((2 * N, 128), jnp.float32),
        ],
    )(h, W1a, W1b)


def _edge_body(p2, q2, rowh, colh, bth, ctab, out,
               acc, ridx, roff, coff, boff, pbuf, qbuf, cbuf, obuf,
               zbuf, sem1, sem2, sem3):
    c = lax.axis_index("c")
    s = lax.axis_index("s")
    e0 = s * EPT
    cN = c * N
    c8 = c * 8

    for r in range(CH, CHP):
        for j in range(UW // 16):
            obuf[r, pl.ds(16 * j, 16)] = jnp.zeros((16,), jnp.float32)

    @pl.loop(0, ZR)
    def _zb(r):
        for j in range(UW // 16):
            zbuf[r, pl.ds(16 * j, 16)] = jnp.zeros((16,), jnp.float32)

    @pl.loop(0, RPT // ZR)
    def _zero(k):
        pltpu.sync_copy(zbuf, acc.at[pl.ds(s * RPT + k * ZR, ZR)])

    plsc.subcore_barrier()

    @pl.loop(0, NCH)
    def _chunk(k):
        base = e0 + k * CH
        pltpu.sync_copy(rowh.at[pl.ds(base, CHP)], ridx)
        pltpu.sync_copy(colh.at[pl.ds(base, CHP)], coff)
        pltpu.sync_copy(bth.at[pl.ds(base, CHP)], boff)
        for j in range(CHP // 16):
            sl = pl.ds(16 * j, 16)
            roff[sl] = ridx[sl] + cN
            coff[sl] = coff[sl] + cN
            boff[sl] = boff[sl] + c8
        d1 = pltpu.async_copy(p2.at[roff], pbuf, sem1)
        d2 = pltpu.async_copy(q2.at[coff], qbuf, sem2)
        d3 = pltpu.async_copy(ctab.at[boff], cbuf, sem3)
        d1.wait()
        d2.wait()
        d3.wait()

        @pl.loop(0, CH)
        def _edge(e):
            wv = cbuf[e, pl.ds(128, 16)]
            for j in range(8):
                sl = pl.ds(16 * j, 16)
                t = pbuf[e, sl] + qbuf[e, sl] + cbuf[e, sl]
                obuf[e, sl] = jnp.maximum(t, 0.0) * wv

        pltpu.sync_copy(obuf, acc.at[ridx], add=True)

    plsc.subcore_barrier()

    @pl.loop(0, RPT // ZR)
    def _wb(k):
        r0 = s * RPT + k * ZR
        pltpu.sync_copy(acc.at[pl.ds(r0, ZR)], zbuf)
        pltpu.sync_copy(zbuf, out.at[c, pl.ds(r0, ZR)])


@functools.cache
def _build_edge_call():
    return functools.partial(
        pl.kernel,
        out_type=jax.ShapeDtypeStruct((2, NP, UW), jnp.float32),
        mesh=plsc.VectorSubcoreMesh(core_axis_name="c", subcore_axis_name="s"),
        scratch_types=[
            pltpu.VMEM_SHARED((NP, UW), jnp.float32),
            pltpu.VMEM((CHP,), jnp.int32),
            pltpu.VMEM((CHP,), jnp.int32),
            pltpu.VMEM((CHP,), jnp.int32),
            pltpu.VMEM((CHP,), jnp.int32),
            pltpu.VMEM((CHP, 128), jnp.float32),
            pltpu.VMEM((CHP, 128), jnp.float32),
            pltpu.VMEM((CHP, 256), jnp.float32),
            pltpu.VMEM((CHP, UW), jnp.float32),
            pltpu.VMEM((ZR, UW), jnp.float32),
            pltpu.SemaphoreType.DMA,
            pltpu.SemaphoreType.DMA,
            pltpu.SemaphoreType.DMA,
        ],
    )(_edge_body)


def _edge_call(*args):
    return _build_edge_call()(*args)


def _post_body(h_ref, u0_ref, u1_ref, w2t_ref, w2b_ref, hn_ref):
    u0 = u0_ref[...]
    u1 = u1_ref[...]
    upd = (jnp.dot(u0, w2t_ref[...], preferred_element_type=jnp.float32)
           + jnp.dot(u1, w2b_ref[...], preferred_element_type=jnp.float32))
    hn_ref[...] = h_ref[...] + upd


def _post(h, u0, u1, W2t, W2b):
    return pl.pallas_call(
        _post_body,
        grid=(NG,),
        in_specs=[
            pl.BlockSpec((NB, 256), lambda i: (i, 0)),
            pl.BlockSpec((NB, UW), lambda i: (i, 0)),
            pl.BlockSpec((NB, UW), lambda i: (i, 0)),
            pl.BlockSpec((128, 256), lambda i: (0, 0)),
            pl.BlockSpec((128, 256), lambda i: (0, 0)),
        ],
        out_specs=pl.BlockSpec((NB, 256), lambda i: (i, 0)),
        out_shape=jax.ShapeDtypeStruct((N, 256), jnp.float32),
    )(h, u0, u1, W2t, W2b)


def _final_body(h_ref, wp1_ref, bp1_ref, wp2_ref, bp2_ref, wfg_ref, bfg_ref,
                props_ref, hsum_ref, fg_ref):
    i = pl.program_id(0)
    hb = h_ref[...]
    z = jnp.maximum(
        jnp.dot(hb, wp1_ref[...], preferred_element_type=jnp.float32)
        + bp1_ref[...], 0.0)
    props_ref[...] = (jnp.dot(z, wp2_ref[...], preferred_element_type=jnp.float32)
                      + bp2_ref[...])

    @pl.when(i == 0)
    def _():
        hsum_ref[...] = jnp.zeros((1, 256), jnp.float32)

    hsum_ref[...] += jnp.sum(hb, axis=0, keepdims=True)

    @pl.when(i == NG - 1)
    def _():
        hbar = hsum_ref[...] * (1.0 / N)
        fg_ref[...] = (jnp.dot(hbar, wfg_ref[...],
                               preferred_element_type=jnp.float32)
                       + bfg_ref[...])


def _final(h, Wp1, bp1r, Wp2, bp2r, Wfgc, bfgr):
    return pl.pallas_call(
        _final_body,
        grid=(NG,),
        in_specs=[
            pl.BlockSpec((NB, 256), lambda i: (i, 0)),
            pl.BlockSpec((256, 128), lambda i: (0, 0)),
            pl.BlockSpec((1, 128), lambda i: (0, 0)),
            pl.BlockSpec((128, 32), lambda i: (0, 0)),
            pl.BlockSpec((1, 32), lambda i: (0, 0)),
            pl.BlockSpec((256, 64), lambda i: (0, 0)),
            pl.BlockSpec((1, 64), lambda i: (0, 0)),
        ],
        out_specs=[
            pl.BlockSpec((NB, 32), lambda i: (i, 0)),
            pl.BlockSpec((1, 256), lambda i: (0, 0)),
            pl.BlockSpec((1, 64), lambda i: (0, 0)),
        ],
        out_shape=[
            jax.ShapeDtypeStruct((N, 32), jnp.float32),
            jax.ShapeDtypeStruct((1, 256), jnp.float32),
            jax.ShapeDtypeStruct((1, 64), jnp.float32),
        ],
    )(h, Wp1, bp1r, Wp2, bp2r, Wfgc, bfgr)


def kernel(x, edge_index, edge_attr, batch, bond_table, atom_table,
           charge_table, hybrid_table, Wn1, bn1, Wn2, bn2, Wa, ba,
           Wp1, bp1, Wp2, bp2, Wfg, bfg):
    pad = jnp.zeros((CHP - CH,), jnp.int32)
    row = jnp.concatenate([edge_index[0], pad])
    col = jnp.concatenate([edge_index[1], pad])

    h, ids = _embed(x, atom_table, charge_table, hybrid_table)
    atom_types = ids.reshape(N)
    ctab = _tables(bond_table, Wn1, bn1, Wa, ba)
    bt = jnp.concatenate([_bt(edge_attr).reshape(E), pad])

    for i in range(3):
        P2, Q2 = _pq(h, Wn1[i, :H], Wn1[i, H:2 * H])
        u = _edge_call(P2, Q2, row, col, bt, ctab[i])
        h = _post(h, u[0, :N], u[1, :N], Wn2[i, :128], Wn2[i, 128:])

    Wfgc = jnp.concatenate([Wfg[j] for j in range(4)], axis=1)
    bfgr = bfg.reshape(1, 64)
    props, _hsum, fgrow = _final(h, Wp1, bp1.reshape(1, 128), Wp2,
                                 bp2.reshape(1, 32), Wfgc, bfgr)
    fg = jnp.broadcast_to(fgrow.reshape(1, 64), (N, 64))
    return (h, props, fg, atom_types)


# speedup vs baseline: 2.9659x; 2.9659x over previous
import functools

import jax
import jax.numpy as jnp
from jax import lax
from jax.experimental import pallas as pl
from jax.experimental.pallas import tpu as pltpu
from jax.experimental.pallas import tpu_sc as plsc

N = 10000
E = 160000
H = 256

NB = 1000
NG = N // NB

NTILES = 16
EPT = E // NTILES
CH = 40
CHP = 48
NCH = EPT // CH
NP = 10112
RPT = NP // NTILES
ZR = 32
UW = 128


def _embed_body(x_ref, at_ref, ct_ref, ht_ref, h_ref, ids_ref):
    xb = x_ref[...]
    a_id = jnp.clip(xb[:, 0:1].astype(jnp.int32), 0, 10)
    c_id = jnp.clip(xb[:, 2:3].astype(jnp.int32) + 3, 0, 6)
    y_id = jnp.clip(xb[:, 1:2].astype(jnp.int32), 0, 7)

    def onehot(ids, k):
        i = lax.broadcasted_iota(jnp.int32, (NB, k), 1)
        return (ids == i).astype(jnp.float32)

    a = onehot(a_id, 11) @ at_ref[...]
    c = onehot(c_id, 7) @ ct_ref[...]
    y = onehot(y_id, 8) @ ht_ref[...]
    h_ref[...] = jnp.concatenate(
        [a, c, y, jnp.zeros((NB, 128), jnp.float32)], axis=1)
    ids_ref[...] = a_id


def _embed(x, atom_table, charge_table, hybrid_table):
    h0, ids = pl.pallas_call(
        _embed_body,
        grid=(NG,),
        in_specs=[
            pl.BlockSpec((NB, 3), lambda i: (i, 0)),
            pl.BlockSpec((11, 64), lambda i: (0, 0)),
            pl.BlockSpec((7, 32), lambda i: (0, 0)),
            pl.BlockSpec((8, 32), lambda i: (0, 0)),
        ],
        out_specs=[
            pl.BlockSpec((NB, 256), lambda i: (i, 0)),
            pl.BlockSpec((NB, 1), lambda i: (i, 0)),
        ],
        out_shape=[
            jax.ShapeDtypeStruct((N, 256), jnp.float32),
            jax.ShapeDtypeStruct((N, 1), jnp.int32),
        ],
    )(x, atom_table, charge_table, hybrid_table)
    return h0, ids


def _tables_body(bt_ref, wn1_ref, bn1_ref, wa_ref, ba_ref,
                 ctabm_ref, bww_ref):
    btab = bt_ref[...]
    for i in range(3):
        C = btab @ wn1_ref[i, 2 * H:, :] + bn1_ref[i:i + 1, :]
        bw = jax.nn.sigmoid(btab @ wa_ref[i] + ba_ref[i:i + 1, :])
        Cw = C * bw
        Cwp = jnp.concatenate([Cw, jnp.zeros((3, 256), jnp.float32)], 0)
        bwp = jnp.broadcast_to(
            jnp.concatenate([bw, jnp.zeros((3, 1), jnp.float32)], 0), (8, 128))
        bww_ref[i] = bwp
        for c in range(2):
            ctabm_ref[i, pl.ds(8 * c, 8)] = Cwp[:, 128 * c:128 * c + 128]


def _tables(bond_table, Wn1, bn1, Wa, ba):
    return pl.pallas_call(
        _tables_body,
        out_shape=[
            jax.ShapeDtypeStruct((3, 16, 128), jnp.float32),
            jax.ShapeDtypeStruct((3, 8, 128), jnp.float32),
        ],
    )(bond_table, Wn1, bn1, Wa, ba)


def _scale_body(p_ref, q_ref, cm_ref, bw_ref, t1_ref, t2_ref):
    bwr = bw_ref[0]
    t1_ref[...] = p_ref[...] * bwr + cm_ref[0]
    t2_ref[...] = q_ref[...] * bwr


def _scale(P2, Q2, ctabm_i, bww_i):
    return pl.pallas_call(
        _scale_body,
        grid=(NG, 2, 5),
        in_specs=[
            pl.BlockSpec((NB, 128), lambda i, c, b: (c * NG + i, 0)),
            pl.BlockSpec((NB, 128), lambda i, c, b: (c * NG + i, 0)),
            pl.BlockSpec((1, 1, 128), lambda i, c, b: (c * 8 + b, 0, 0)),
            pl.BlockSpec((1, 1, 128), lambda i, c, b: (b, 0, 0)),
        ],
        out_specs=[
            pl.BlockSpec((NB, 128), lambda i, c, b: ((c * 5 + b) * NG + i, 0)),
            pl.BlockSpec((NB, 128), lambda i, c, b: ((c * 5 + b) * NG + i, 0)),
        ],
        out_shape=[
            jax.ShapeDtypeStruct((10 * N, 128), jnp.float32),
            jax.ShapeDtypeStruct((10 * N, 128), jnp.float32),
        ],
    )(P2, Q2, ctabm_i.reshape(16, 1, 128), bww_i.reshape(8, 1, 128))


def _bt_body(ea_ref, bt_ref):
    bt_ref[...] = jnp.clip(ea_ref[...][:, 0:1].astype(jnp.int32), 0, 4)


def _bt(edge_attr):
    return pl.pallas_call(
        _bt_body,
        grid=(E // NB,),
        in_specs=[pl.BlockSpec((NB, 4), lambda i: (i, 0))],
        out_specs=pl.BlockSpec((NB, 1), lambda i: (i, 0)),
        out_shape=jax.ShapeDtypeStruct((E, 1), jnp.int32),
    )(edge_attr)


def _pq_body(h_ref, wa_ref, wb_ref, p_ref, q_ref):
    hb = h_ref[...]
    p_ref[...] = jnp.dot(hb, wa_ref[...], preferred_element_type=jnp.float32)
    q_ref[...] = jnp.dot(hb, wb_ref[...], preferred_element_type=jnp.float32)


def _pq(h, W1a, W1b):
    return pl.pallas_call(
        _pq_body,
        grid=(NG, 2),
        in_specs=[
            pl.BlockSpec((NB, 256), lambda i, j: (i, 0)),
            pl.BlockSpec((256, 128), lambda i, j: (0, j)),
            pl.BlockSpec((256, 128), lambda i, j: (0, j)),
        ],
        out_specs=[
            pl.BlockSpec((NB, 128), lambda i, j: (j * NG + i, 0)),
            pl.BlockSpec((NB, 128), lambda i, j: (j * NG + i, 0)),
        ],
        out_shape=[
            jax.ShapeDtypeStruct((2 * N, 128), jnp.float32),
            jax.ShapeDtypeStruct((2 * N, 128), jnp.float32),
        ],
    )(h, W1a, W1b)


def _edge_body(t1, t2, rowh, colh, bth, out,
               acc,
               rA, roA, coA, btA, pA, qA, oA, srA,
               rB, roB, coB, btB, pB, qB, oB, srB,
               zbuf, sgA, sgB, ssA, ssB):
    c = lax.axis_index("c")
    s = lax.axis_index("s")
    e0 = s * EPT
    c5N = c * (5 * N)

    BUFS = ((rA, roA, coA, btA, pA, qA, oA, srA, sgA, ssA),
            (rB, roB, coB, btB, pB, qB, oB, srB, sgB, ssB))

    for OB in (oA, oB):
        for r in range(CH, CHP):
            for j in range(UW // 16):
                OB[r, pl.ds(16 * j, 16)] = jnp.zeros((16,), jnp.float32)

    @pl.loop(0, ZR)
    def _zb(r):
        for j in range(UW // 16):
            zbuf[r, pl.ds(16 * j, 16)] = jnp.zeros((16,), jnp.float32)

    for i in range(RPT // ZR):
        pltpu.async_copy(zbuf, acc.at[pl.ds(s * RPT + i * ZR, ZR)], sgA)
    for i in range(RPT // ZR):
        pltpu.make_async_copy(
            zbuf, acc.at[pl.ds(s * RPT + i * ZR, ZR)], sgA).wait()

    plsc.subcore_barrier()

    def load_and_fire(k, buf):
        R, RO, CO, BT, PB, QB, OB, SR, sg, ss = buf
        base = e0 + k * CH
        pltpu.sync_copy(rowh.at[pl.ds(base, CHP)], R)
        pltpu.sync_copy(colh.at[pl.ds(base, CHP)], CO)
        pltpu.sync_copy(bth.at[pl.ds(base, CHP)], BT)
        for j in range(CHP // 16):
            sl = pl.ds(16 * j, 16)
            t = BT[sl] * N + c5N
            RO[sl] = t + R[sl]
            CO[sl] = t + CO[sl]
        pltpu.async_copy(t1.at[RO], PB, sg)
        pltpu.async_copy(t2.at[CO], QB, sg)

    def wait_gathers(buf):
        R, RO, CO, BT, PB, QB, OB, SR, sg, ss = buf
        pltpu.make_async_copy(t1.at[RO], PB, sg).wait()
        pltpu.make_async_copy(t2.at[CO], QB, sg).wait()

    def compute(buf):
        R, RO, CO, BT, PB, QB, OB, SR, sg, ss = buf

        @pl.loop(0, CH)
        def _edge(e):
            for j in range(8):
                sl = pl.ds(16 * j, 16)
                OB[e, sl] = jnp.maximum(PB[e, sl] + QB[e, sl], 0.0)

    def fire_scatter(buf):
        R, RO, CO, BT, PB, QB, OB, SR, sg, ss = buf
        for j in range(CHP // 16):
            sl = pl.ds(16 * j, 16)
            SR[sl] = R[sl]
        pltpu.async_copy(OB, acc.at[SR], ss, add=True)

    def wait_scatter(buf):
        R, RO, CO, BT, PB, QB, OB, SR, sg, ss = buf
        pltpu.make_async_copy(OB, acc.at[SR], ss).wait()

    load_and_fire(0, BUFS[0])

    @pl.loop(0, (NCH + 1) // 2)
    def _pair(m):
        for b in range(2):
            k = 2 * m + b

            @pl.when(k < NCH)
            def _():
                @pl.when(k + 1 < NCH)
                def _():
                    load_and_fire(k + 1, BUFS[1 - b])

                wait_gathers(BUFS[b])
                compute(BUFS[b])

                @pl.when(k >= 1)
                def _():
                    wait_scatter(BUFS[1 - b])

                fire_scatter(BUFS[b])

    wait_scatter(BUFS[(NCH - 1) % 2])

    plsc.subcore_barrier()

    @pl.loop(0, RPT // ZR)
    def _wb(k):
        r0 = s * RPT + k * ZR
        pltpu.sync_copy(acc.at[pl.ds(r0, ZR)], zbuf)
        pltpu.sync_copy(zbuf, out.at[c, pl.ds(r0, ZR)])


@functools.cache
def _build_edge_call():
    ibuf = lambda: pltpu.VMEM((CHP,), jnp.int32)
    fbuf = lambda: pltpu.VMEM((CHP, 128), jnp.float32)
    return functools.partial(
        pl.kernel,
        out_type=jax.ShapeDtypeStruct((2, NP, UW), jnp.float32),
        mesh=plsc.VectorSubcoreMesh(core_axis_name="c", subcore_axis_name="s"),
        scratch_types=[
            pltpu.VMEM_SHARED((NP, UW), jnp.float32),
            ibuf(), ibuf(), ibuf(), ibuf(), fbuf(), fbuf(), fbuf(), ibuf(),
            ibuf(), ibuf(), ibuf(), ibuf(), fbuf(), fbuf(), fbuf(), ibuf(),
            pltpu.VMEM((ZR, UW), jnp.float32),
            pltpu.SemaphoreType.DMA,
            pltpu.SemaphoreType.DMA,
            pltpu.SemaphoreType.DMA,
            pltpu.SemaphoreType.DMA,
        ],
    )(_edge_body)


def _edge_call(*args):
    return _build_edge_call()(*args)


def _post_body(h_ref, u0_ref, u1_ref, w2t_ref, w2b_ref, hn_ref):
    u0 = u0_ref[...]
    u1 = u1_ref[...]
    upd = (jnp.dot(u0, w2t_ref[...], preferred_element_type=jnp.float32)
           + jnp.dot(u1, w2b_ref[...], preferred_element_type=jnp.float32))
    hn_ref[...] = h_ref[...] + upd


def _post(h, u0, u1, W2t, W2b):
    return pl.pallas_call(
        _post_body,
        grid=(NG,),
        in_specs=[
            pl.BlockSpec((NB, 256), lambda i: (i, 0)),
            pl.BlockSpec((NB, UW), lambda i: (i, 0)),
            pl.BlockSpec((NB, UW), lambda i: (i, 0)),
            pl.BlockSpec((128, 256), lambda i: (0, 0)),
            pl.BlockSpec((128, 256), lambda i: (0, 0)),
        ],
        out_specs=pl.BlockSpec((NB, 256), lambda i: (i, 0)),
        out_shape=jax.ShapeDtypeStruct((N, 256), jnp.float32),
    )(h, u0, u1, W2t, W2b)


def _final_body(h_ref, wp1_ref, bp1_ref, wp2_ref, bp2_ref, wfg_ref, bfg_ref,
                props_ref, hsum_ref, fg_ref):
    i = pl.program_id(0)
    hb = h_ref[...]
    z = jnp.maximum(
        jnp.dot(hb, wp1_ref[...], preferred_element_type=jnp.float32)
        + bp1_ref[...], 0.0)
    props_ref[...] = (jnp.dot(z, wp2_ref[...], preferred_element_type=jnp.float32)
                      + bp2_ref[...])

    @pl.when(i == 0)
    def _():
        hsum_ref[...] = jnp.zeros((1, 256), jnp.float32)

    hsum_ref[...] += jnp.sum(hb, axis=0, keepdims=True)

    @pl.when(i == NG - 1)
    def _():
        hbar = hsum_ref[...] * (1.0 / N)
        fg_ref[...] = (jnp.dot(hbar, wfg_ref[...],
                               preferred_element_type=jnp.float32)
                       + bfg_ref[...])


def _final(h, Wp1, bp1r, Wp2, bp2r, Wfgc, bfgr):
    return pl.pallas_call(
        _final_body,
        grid=(NG,),
        in_specs=[
            pl.BlockSpec((NB, 256), lambda i: (i, 0)),
            pl.BlockSpec((256, 128), lambda i: (0, 0)),
            pl.BlockSpec((1, 128), lambda i: (0, 0)),
            pl.BlockSpec((128, 32), lambda i: (0, 0)),
            pl.BlockSpec((1, 32), lambda i: (0, 0)),
            pl.BlockSpec((256, 64), lambda i: (0, 0)),
            pl.BlockSpec((1, 64), lambda i: (0, 0)),
        ],
        out_specs=[
            pl.BlockSpec((NB, 32), lambda i: (i, 0)),
            pl.BlockSpec((1, 256), lambda i: (0, 0)),
            pl.BlockSpec((1, 64), lambda i: (0, 0)),
        ],
        out_shape=[
            jax.ShapeDtypeStruct((N, 32), jnp.float32),
            jax.ShapeDtypeStruct((1, 256), jnp.float32),
            jax.ShapeDtypeStruct((1, 64), jnp.float32),
        ],
    )(h, Wp1, bp1r, Wp2, bp2r, Wfgc, bfgr)


def kernel(x, edge_index, edge_attr, batch, bond_table, atom_table,
           charge_table, hybrid_table, Wn1, bn1, Wn2, bn2, Wa, ba,
           Wp1, bp1, Wp2, bp2, Wfg, bfg):
    pad = jnp.zeros((CHP - CH,), jnp.int32)
    row = jnp.concatenate([edge_index[0], pad])
    col = jnp.concatenate([edge_index[1], pad])

    h, ids = _embed(x, atom_table, charge_table, hybrid_table)
    atom_types = ids.reshape(N)
    ctabm, bww = _tables(bond_table, Wn1, bn1, Wa, ba)
    bt = jnp.concatenate([_bt(edge_attr).reshape(E), pad])

    for i in range(3):
        P2, Q2 = _pq(h, Wn1[i, :H], Wn1[i, H:2 * H])
        T1, T2 = _scale(P2, Q2, ctabm[i], bww[i])
        u = _edge_call(T1, T2, row, col, bt)
        h = _post(h, u[0, :N], u[1, :N], Wn2[i, :128], Wn2[i, 128:])

    Wfgc = jnp.concatenate([Wfg[j] for j in range(4)], axis=1)
    bfgr = bfg.reshape(1, 64)
    props, _hsum, fgrow = _final(h, Wp1, bp1.reshape(1, 128), Wp2,
                                 bp2.reshape(1, 32), Wfgc, bfgr)
    fg = jnp.broadcast_to(fgrow.reshape(1, 64), (N, 64))
    return (h, props, fg, atom_types)
